# baseline (speedup 1.0000x reference)
import jax
import jax.numpy as jnp
from jax.experimental import pallas as pl
from jax.experimental.pallas import tpu as pltpu

IN_C = 96
OUT_C = 96
BREADTH = 8
H_BLK = 16
N_CHUNKS = 7


def _fused_kernel(x_ref, sel_ref, w_ref, b_ref, o_ref):
    n = H_BLK * 224
    x = x_ref[0].astype(jnp.bfloat16).reshape(IN_C, n)
    s = sel_ref[0].reshape(BREADTH, n)
    m = jnp.max(s, axis=0, keepdims=True)
    e = jnp.exp(s - m)
    gate = (e / jnp.sum(e, axis=0, keepdims=True)).astype(jnp.bfloat16)

    w = w_ref[...]
    ch = n // N_CHUNKS
    y_parts = []
    for k in range(N_CHUNKS):
        xk = x[:, k * ch:(k + 1) * ch]
        zk = jnp.concatenate(
            [xk * gate[si:si + 1, k * ch:(k + 1) * ch]
             for si in range(BREADTH)], axis=0)
        y_parts.append(jax.lax.dot_general(
            w, zk, (((1,), (0,)), ((), ())),
            preferred_element_type=jnp.float32))
    y = jnp.concatenate(y_parts, axis=1)

    o_ref[0] = (y + b_ref[...]).astype(jnp.bfloat16).reshape(
        OUT_C, H_BLK, 224).astype(jnp.float32)


def kernel(input, selector, weight, bias):
    b, c, h, w_dim = input.shape
    w = jnp.transpose(weight[:, :, :, 0, 0], (0, 2, 1)).reshape(
        OUT_C, BREADTH * IN_C).astype(jnp.bfloat16)
    b2 = bias.reshape(OUT_C, 1)

    grid = (b, h // H_BLK)
    out = pl.pallas_call(
        _fused_kernel,
        grid=grid,
        in_specs=[
            pl.BlockSpec((1, IN_C, H_BLK, w_dim), lambda i, j: (i, 0, j, 0)),
            pl.BlockSpec((1, BREADTH, H_BLK, w_dim), lambda i, j: (i, 0, j, 0)),
            pl.BlockSpec((OUT_C, BREADTH * IN_C), lambda i, j: (0, 0)),
            pl.BlockSpec((OUT_C, 1), lambda i, j: (0, 0)),
        ],
        out_specs=pl.BlockSpec((1, OUT_C, H_BLK, w_dim), lambda i, j: (i, 0, j, 0)),
        out_shape=jax.ShapeDtypeStruct((b, OUT_C, h, w_dim), jnp.float32),
        compiler_params=pltpu.CompilerParams(
            dimension_semantics=("parallel", "parallel")),
    )(input, selector, w, b2)
    return out


# speedup vs baseline: 1.2122x; 1.2122x over previous
import jax
import jax.numpy as jnp
from jax.experimental import pallas as pl
from jax.experimental.pallas import tpu as pltpu

IN_C = 96
OUT_C = 96
BREADTH = 8
H_BLK = 56
N_CHUNKS = 7


def _fused_kernel(x_ref, sel_ref, w_ref, b_ref, o_ref):
    n = H_BLK * 224
    w = w_ref[...]
    bcol = b_ref[...]
    x = x_ref[0].astype(jnp.bfloat16).reshape(IN_C, n)
    s = sel_ref[0].reshape(BREADTH, n)
    m = jnp.max(s, axis=0, keepdims=True)
    e = jnp.exp(s - m)
    gate = (e / jnp.sum(e, axis=0, keepdims=True)).astype(jnp.bfloat16)

    ch = n // N_CHUNKS
    y_parts = []
    for k in range(N_CHUNKS):
        xk = x[:, k * ch:(k + 1) * ch]
        zk = jnp.concatenate(
            [xk * gate[si:si + 1, k * ch:(k + 1) * ch]
             for si in range(BREADTH)], axis=0)
        y_parts.append(jax.lax.dot_general(
            w, zk, (((1,), (0,)), ((), ())),
            preferred_element_type=jnp.float32))
    y = jnp.concatenate(y_parts, axis=1)

    o_ref[0] = (y + bcol).astype(jnp.bfloat16).reshape(
        OUT_C, H_BLK, 224).astype(jnp.float32)


def kernel(input, selector, weight, bias):
    b, c, h, w_dim = input.shape
    w = jnp.transpose(weight[:, :, :, 0, 0], (0, 2, 1)).reshape(
        OUT_C, BREADTH * IN_C).astype(jnp.bfloat16)
    b2 = bias.reshape(OUT_C, 1)
    grid = (b, h // H_BLK)
    out = pl.pallas_call(
        _fused_kernel,
        grid=grid,
        in_specs=[
            pl.BlockSpec((1, IN_C, H_BLK, w_dim), lambda i, j: (i, 0, j, 0)),
            pl.BlockSpec((1, BREADTH, H_BLK, w_dim), lambda i, j: (i, 0, j, 0)),
            pl.BlockSpec((OUT_C, BREADTH * IN_C), lambda i, j: (0, 0)),
            pl.BlockSpec((OUT_C, 1), lambda i, j: (0, 0)),
        ],
        out_specs=pl.BlockSpec((1, OUT_C, H_BLK, w_dim), lambda i, j: (i, 0, j, 0)),
        out_shape=jax.ShapeDtypeStruct((b, OUT_C, h, w_dim), jnp.float32),
        compiler_params=pltpu.CompilerParams(
            dimension_semantics=("parallel", "parallel")),
    )(input, selector, w, b2)
    return out
